# Initial kernel scaffold; baseline (speedup 1.0000x reference)
import jax
import jax.numpy as jnp
from jax.experimental import pallas as pl


def kernel(inputs, codebook):
    raise NotImplementedError("write your pallas kernel here")


# speedup vs baseline: 1.2555x; 1.2555x over previous
import functools

import jax
import jax.numpy as jnp
from jax import lax
from jax.experimental import pallas as pl
from jax.experimental.pallas import tpu as pltpu
from jax.experimental.pallas import tpu_sc as plsc

K = 8192
D = 256
BT = 512


_CHUNK = 2816


def _argmin_body(x_ref, cb_ref, x2_ref, e2_ref, idx_ref, commit_ref):
    t = pl.program_id(0)
    dots = lax.dot_general(x_ref[...].astype(jnp.bfloat16),
                           cb_ref[...].astype(jnp.bfloat16),
                           (((1,), (1,)), ((), ())),
                           preferred_element_type=jnp.float32)
    dist = x2_ref[...] - 2.0 * dots + e2_ref[...]
    kk = dist.shape[1]
    bounds = [(lo, min(lo + _CHUNK, kk)) for lo in range(0, kk, _CHUNK)]
    rv_cmp = None
    ri = None
    rexact = None
    for lo, hi in bounds:
        dd = dist[:, lo:hi]
        lmin = jnp.min(dd, axis=1, keepdims=True)
        col = lax.broadcasted_iota(jnp.int32, dd.shape, 1)
        lidx = jnp.min(jnp.where(dd == lmin, col, kk),
                       axis=1, keepdims=True) + lo
        if rv_cmp is None:
            rv_cmp, ri, rexact = lmin.astype(jnp.bfloat16), lidx, lmin
        else:
            rvf = rv_cmp.astype(jnp.float32)
            take = lmin < rvf
            ri = jnp.where(take, lidx, ri)
            rexact = jnp.where(take, lmin, rexact)
            rv_cmp = jnp.where(take, lmin, rvf).astype(jnp.bfloat16)
    idx_ref[...] = ri
    s = jnp.sum(rexact)

    @pl.when(t == 0)
    def _():
        commit_ref[0, 0] = s

    @pl.when(t > 0)
    def _():
        commit_ref[0, 0] = commit_ref[0, 0] + s


def _argmin_call(x, cb, x2, e2, bt=BT, interpret=False):
    ntok, d = x.shape
    kk = cb.shape[0]
    nt = ntok // bt
    return pl.pallas_call(
        _argmin_body,
        grid=(nt,),
        in_specs=[
            pl.BlockSpec((bt, d), lambda t: (t, 0)),
            pl.BlockSpec((kk, d), lambda t: (0, 0)),
            pl.BlockSpec((bt, 1), lambda t: (t, 0)),
            pl.BlockSpec((1, kk), lambda t: (0, 0)),
        ],
        out_specs=[
            pl.BlockSpec((bt, 1), lambda t: (t, 0)),
            pl.BlockSpec((1, 1), lambda t: (0, 0),
                         memory_space=pltpu.SMEM),
        ],
        out_shape=[
            jax.ShapeDtypeStruct((ntok, 1), jnp.int32),
            jax.ShapeDtypeStruct((1, 1), jnp.float32),
        ],
        interpret=interpret,
    )(x, cb, x2, e2)


@functools.cache
def _make_gather(batch):
    info = plsc.get_sparse_core_info()
    nw = info.num_cores * info.num_subcores
    bpw = batch // nw
    ch = min(bpw, 128)
    nch = bpw // ch
    mesh = plsc.VectorSubcoreMesh(core_axis_name="c", subcore_axis_name="s")

    @functools.partial(
        pl.kernel, mesh=mesh,
        out_type=jax.ShapeDtypeStruct((batch, D), jnp.float32),
        scratch_types=[
            pltpu.VMEM((nch, ch), jnp.int32),
            pltpu.VMEM((ch, D), jnp.float32),
            pltpu.SemaphoreType.DMA,
        ],
    )
    def gather_kernel(table_hbm, idx_hbm, out_hbm, idx_v, rows_v, sem):
        wid = lax.axis_index("s") * info.num_cores + lax.axis_index("c")
        base = wid * bpw
        for ci in range(nch):
            pltpu.sync_copy(idx_hbm.at[pl.ds(base + ci * ch, ch)],
                            idx_v.at[ci])
            pltpu.async_copy(table_hbm.at[idx_v.at[ci]], rows_v, sem).wait()
            pltpu.sync_copy(rows_v, out_hbm.at[pl.ds(base + ci * ch, ch)])

    return gather_kernel


def kernel(inputs, codebook):
    b, c, h, w = inputs.shape
    n = h * w
    batch = b * n
    x = jnp.transpose(inputs, (0, 2, 3, 1)).reshape(batch, c)
    x2 = jnp.sum(x * x, axis=-1, keepdims=True)
    e2 = jnp.sum(codebook * codebook, axis=-1).reshape(1, codebook.shape[0])
    idx2d, csum = _argmin_call(x, codebook, x2, e2)
    q = _make_gather(batch)(codebook, idx2d.reshape(batch))
    st = x + (q - x)
    quantized = jnp.transpose(st.reshape(b, h, w, c), (0, 3, 1, 2))
    commit = csum[0, 0] / (batch * c)
    return quantized, commit


# speedup vs baseline: 1.3558x; 1.0799x over previous
import functools

import jax
import jax.numpy as jnp
from jax import lax
from jax.experimental import pallas as pl
from jax.experimental.pallas import tpu as pltpu
from jax.experimental.pallas import tpu_sc as plsc

K = 8192
D = 256
BT = 512


_CHUNK = 2816


def _argmin_body(xm2_ref, cb_ref, x2_ref, e2_ref, colf_ref, idx_ref, commit_ref):
    t = pl.program_id(0)
    dots2 = lax.dot_general(xm2_ref[...], cb_ref[...],
                            (((1,), (1,)), ((), ())),
                            preferred_element_type=jnp.float32)
    dist = x2_ref[...] + dots2 + e2_ref[...]
    kk = dist.shape[1]
    bounds = [(lo, min(lo + _CHUNK, kk)) for lo in range(0, kk, _CHUNK)]
    rv_cmp = None
    ri = None
    rexact = None
    for lo, hi in bounds:
        dd = dist[:, lo:hi]
        lmin = jnp.min(dd, axis=1, keepdims=True)
        colf = colf_ref[...][:, lo:hi]
        lidx = jnp.min(jnp.where(dd == lmin, colf, float(kk)),
                       axis=1, keepdims=True)
        if rv_cmp is None:
            rv_cmp, ri, rexact = lmin.astype(jnp.bfloat16), lidx, lmin
        else:
            rvf = rv_cmp.astype(jnp.float32)
            take = lmin < rvf
            ri = jnp.where(take, lidx, ri)
            rexact = jnp.where(take, lmin, rexact)
            rv_cmp = jnp.where(take, lmin, rvf).astype(jnp.bfloat16)
    idx_ref[...] = ri.astype(jnp.int32)
    s = jnp.sum(rexact)

    @pl.when(t == 0)
    def _():
        commit_ref[0, 0] = s

    @pl.when(t > 0)
    def _():
        commit_ref[0, 0] = commit_ref[0, 0] + s


def _argmin_call(xm2, cb, x2, e2, bt=BT, interpret=False):
    ntok, d = xm2.shape
    kk = cb.shape[0]
    nt = ntok // bt
    return pl.pallas_call(
        _argmin_body,
        grid=(nt,),
        in_specs=[
            pl.BlockSpec((bt, d), lambda t: (t, 0)),
            pl.BlockSpec((kk, d), lambda t: (0, 0)),
            pl.BlockSpec((bt, 1), lambda t: (t, 0)),
            pl.BlockSpec((1, kk), lambda t: (0, 0)),
            pl.BlockSpec((1, kk), lambda t: (0, 0)),
        ],
        out_specs=[
            pl.BlockSpec((bt, 1), lambda t: (t, 0)),
            pl.BlockSpec((1, 1), lambda t: (0, 0),
                         memory_space=pltpu.SMEM),
        ],
        out_shape=[
            jax.ShapeDtypeStruct((ntok, 1), jnp.int32),
            jax.ShapeDtypeStruct((1, 1), jnp.float32),
        ],
        interpret=interpret,
    )(xm2, cb, x2, e2, jnp.arange(kk, dtype=jnp.float32).reshape(1, kk))


@functools.cache
def _make_gather(batch):
    info = plsc.get_sparse_core_info()
    nw = info.num_cores * info.num_subcores
    bpw = batch // nw
    ch = min(bpw, 128)
    nch = bpw // ch
    mesh = plsc.VectorSubcoreMesh(core_axis_name="c", subcore_axis_name="s")

    @functools.partial(
        pl.kernel, mesh=mesh,
        out_type=jax.ShapeDtypeStruct((batch, D), jnp.float32),
        scratch_types=[
            pltpu.VMEM((nch, ch), jnp.int32),
            pltpu.VMEM((ch, D), jnp.float32),
            pltpu.SemaphoreType.DMA,
        ],
    )
    def gather_kernel(table_hbm, idx_hbm, out_hbm, idx_v, rows_v, sem):
        wid = lax.axis_index("s") * info.num_cores + lax.axis_index("c")
        base = wid * bpw
        for ci in range(nch):
            pltpu.sync_copy(idx_hbm.at[pl.ds(base + ci * ch, ch)],
                            idx_v.at[ci])
            pltpu.async_copy(table_hbm.at[idx_v.at[ci]], rows_v, sem).wait()
            pltpu.sync_copy(rows_v, out_hbm.at[pl.ds(base + ci * ch, ch)])

    return gather_kernel


def kernel(inputs, codebook):
    b, c, h, w = inputs.shape
    n = h * w
    batch = b * n
    x = jnp.transpose(inputs, (0, 2, 3, 1)).reshape(batch, c)
    x2 = jnp.sum(x * x, axis=-1, keepdims=True)
    e2 = jnp.sum(codebook * codebook, axis=-1).reshape(1, codebook.shape[0])
    xm2 = (-2.0 * x).astype(jnp.bfloat16)
    cbb = codebook.astype(jnp.bfloat16)
    idx2d, csum = _argmin_call(xm2, cbb, x2, e2)
    q = _make_gather(batch)(codebook, idx2d.reshape(batch))
    st = x + (q - x)
    quantized = jnp.transpose(st.reshape(b, h, w, c), (0, 3, 1, 2))
    commit = csum[0, 0] / (batch * c)
    return quantized, commit


# speedup vs baseline: 1.4871x; 1.0968x over previous
import functools

import jax
import jax.numpy as jnp
from jax import lax
from jax.experimental import pallas as pl
from jax.experimental.pallas import tpu as pltpu
from jax.experimental.pallas import tpu_sc as plsc

K = 8192
D = 256
BT = 512


_CHUNK = 2816


def _argmin_body(xm2_ref, cb_ref, x2_ref, e2_ref, colf_ref, idx_ref, commit_ref):
    t = pl.program_id(0)
    dots2 = lax.dot_general(xm2_ref[...], cb_ref[...],
                            (((1,), (1,)), ((), ())),
                            preferred_element_type=jnp.float32)
    dist = x2_ref[...] + dots2 + e2_ref[...]
    kk = dist.shape[1]
    bounds = [(lo, min(lo + _CHUNK, kk)) for lo in range(0, kk, _CHUNK)]
    rv_cmp = None
    ri = None
    rexact = None
    for lo, hi in bounds:
        dd = dist[:, lo:hi]
        lmin = jnp.min(dd, axis=1, keepdims=True)
        colf = colf_ref[...][:, lo:hi]
        lidx = jnp.min(jnp.where(dd == lmin, colf, float(kk)),
                       axis=1, keepdims=True)
        if rv_cmp is None:
            rv_cmp, ri, rexact = lmin.astype(jnp.bfloat16), lidx, lmin
        else:
            rvf = rv_cmp.astype(jnp.float32)
            take = lmin < rvf
            ri = jnp.where(take, lidx, ri)
            rexact = jnp.where(take, lmin, rexact)
            rv_cmp = jnp.where(take, lmin, rvf).astype(jnp.bfloat16)
    idx_ref[...] = ri.astype(jnp.int32)
    s = jnp.sum(rexact)

    @pl.when(t == 0)
    def _():
        commit_ref[0, 0] = s

    @pl.when(t > 0)
    def _():
        commit_ref[0, 0] = commit_ref[0, 0] + s


def _argmin_call(xm2, cb, x2, e2, bt=BT, interpret=False):
    ntok, d = xm2.shape
    kk = cb.shape[0]
    nt = ntok // bt
    return pl.pallas_call(
        _argmin_body,
        grid=(nt,),
        in_specs=[
            pl.BlockSpec((bt, d), lambda t: (t, 0)),
            pl.BlockSpec((kk, d), lambda t: (0, 0)),
            pl.BlockSpec((bt, 1), lambda t: (t, 0)),
            pl.BlockSpec((1, kk), lambda t: (0, 0)),
            pl.BlockSpec((1, kk), lambda t: (0, 0)),
        ],
        out_specs=[
            pl.BlockSpec((bt, 1), lambda t: (t, 0)),
            pl.BlockSpec((1, 1), lambda t: (0, 0),
                         memory_space=pltpu.SMEM),
        ],
        out_shape=[
            jax.ShapeDtypeStruct((ntok, 1), jnp.int32),
            jax.ShapeDtypeStruct((1, 1), jnp.float32),
        ],
        interpret=interpret,
    )(xm2, cb, x2, e2, jnp.arange(kk, dtype=jnp.float32).reshape(1, kk))


@functools.cache
def _make_gather(batch):
    info = plsc.get_sparse_core_info()
    nw = info.num_cores * info.num_subcores
    bpw = batch // nw
    ch = min(bpw, 128)
    nch = bpw // ch
    mesh = plsc.VectorSubcoreMesh(core_axis_name="c", subcore_axis_name="s")

    @functools.partial(
        pl.kernel, mesh=mesh,
        out_type=jax.ShapeDtypeStruct((batch, D), jnp.float32),
        scratch_types=[
            pltpu.VMEM((nch, ch), jnp.int32),
            pltpu.VMEM((ch, D), jnp.float32),
            pltpu.SemaphoreType.DMA,
        ],
    )
    def gather_kernel(table_hbm, idx_hbm, out_hbm, idx_v, rows_v, sem):
        wid = lax.axis_index("s") * info.num_cores + lax.axis_index("c")
        base = wid * bpw
        for ci in range(nch):
            pltpu.sync_copy(idx_hbm.at[pl.ds(base + ci * ch, ch)],
                            idx_v.at[ci])
            pltpu.async_copy(table_hbm.at[idx_v.at[ci]], rows_v, sem).wait()
            pltpu.sync_copy(rows_v, out_hbm.at[pl.ds(base + ci * ch, ch)])

    return gather_kernel


def kernel(inputs, codebook):
    b, c, h, w = inputs.shape
    n = h * w
    batch = b * n
    x = jnp.transpose(inputs, (0, 2, 3, 1)).reshape(batch, c)
    x2 = jnp.sum(x * x, axis=-1, keepdims=True)
    e2 = jnp.sum(codebook * codebook, axis=-1).reshape(1, codebook.shape[0])
    xm2 = (-2.0 * x).astype(jnp.bfloat16)
    cbb = codebook.astype(jnp.bfloat16)
    idx2d, csum = _argmin_call(xm2, cbb, x2, e2)
    q = _make_gather(batch)(codebook, idx2d.reshape(batch))
    quantized = jnp.transpose(q.reshape(b, h, w, c), (0, 3, 1, 2))
    commit = csum[0, 0] / (batch * c)
    return quantized, commit


# speedup vs baseline: 1.6867x; 1.1342x over previous
import functools

import jax
import jax.numpy as jnp
from jax import lax
from jax.experimental import pallas as pl
from jax.experimental.pallas import tpu as pltpu
from jax.experimental.pallas import tpu_sc as plsc

K = 8192
D = 256
BT = 512


_CHUNK = 2816


def _argmin_body(xm2_ref, cb_ref, x2_ref, e2_ref, idx_ref, commit_ref):
    t = pl.program_id(0)
    dots2 = lax.dot_general(xm2_ref[...], cb_ref[...],
                            (((1,), (1,)), ((), ())),
                            preferred_element_type=jnp.float32)
    kk = dots2.shape[1]
    x2v = x2_ref[...]
    e2v = e2_ref[...]
    nl = 128
    bounds = [(lo, min(lo + _CHUNK, kk)) for lo in range(0, kk, _CHUNK)]
    rv_cmp = None
    ri = None
    rexact = None
    for lo, hi in bounds:
        rv = rj = None
        for j in range((hi - lo) // nl):
            s = lo + j * nl
            ddj = x2v + dots2[:, s:s + nl] + e2v[:, s:s + nl]
            if rv is None:
                rv, rj = ddj, jnp.zeros_like(ddj)
            else:
                m = ddj < rv
                rv = jnp.where(m, ddj, rv)
                rj = jnp.where(m, jnp.float32(j), rj)
        lmin = jnp.min(rv, axis=1, keepdims=True)
        lane = lax.broadcasted_iota(jnp.int32, rv.shape, 1).astype(jnp.float32)
        gidx = rj * float(nl) + lane + float(lo)
        lidx = jnp.min(jnp.where(rv == lmin, gidx, float(kk)),
                       axis=1, keepdims=True)
        if rv_cmp is None:
            rv_cmp, ri, rexact = lmin.astype(jnp.bfloat16), lidx, lmin
        else:
            rvf = rv_cmp.astype(jnp.float32)
            take = lmin < rvf
            ri = jnp.where(take, lidx, ri)
            rexact = jnp.where(take, lmin, rexact)
            rv_cmp = jnp.where(take, lmin, rvf).astype(jnp.bfloat16)
    idx_ref[...] = ri.astype(jnp.int32)
    s = jnp.sum(rexact)

    @pl.when(t == 0)
    def _():
        commit_ref[0, 0] = s

    @pl.when(t > 0)
    def _():
        commit_ref[0, 0] = commit_ref[0, 0] + s


def _argmin_call(xm2, cb, x2, e2, bt=BT, interpret=False):
    ntok, d = xm2.shape
    kk = cb.shape[0]
    nt = ntok // bt
    return pl.pallas_call(
        _argmin_body,
        grid=(nt,),
        in_specs=[
            pl.BlockSpec((bt, d), lambda t: (t, 0)),
            pl.BlockSpec((kk, d), lambda t: (0, 0)),
            pl.BlockSpec((bt, 1), lambda t: (t, 0)),
            pl.BlockSpec((1, kk), lambda t: (0, 0)),
        ],
        out_specs=[
            pl.BlockSpec((bt, 1), lambda t: (t, 0)),
            pl.BlockSpec((1, 1), lambda t: (0, 0),
                         memory_space=pltpu.SMEM),
        ],
        out_shape=[
            jax.ShapeDtypeStruct((ntok, 1), jnp.int32),
            jax.ShapeDtypeStruct((1, 1), jnp.float32),
        ],
        interpret=interpret,
    )(xm2, cb, x2, e2)


@functools.cache
def _make_gather(batch):
    info = plsc.get_sparse_core_info()
    nw = info.num_cores * info.num_subcores
    bpw = batch // nw
    ch = min(bpw, 128)
    nch = bpw // ch
    mesh = plsc.VectorSubcoreMesh(core_axis_name="c", subcore_axis_name="s")

    @functools.partial(
        pl.kernel, mesh=mesh,
        out_type=jax.ShapeDtypeStruct((batch, D), jnp.float32),
        scratch_types=[
            pltpu.VMEM((nch, ch), jnp.int32),
            pltpu.VMEM((ch, D), jnp.float32),
            pltpu.SemaphoreType.DMA,
        ],
    )
    def gather_kernel(table_hbm, idx_hbm, out_hbm, idx_v, rows_v, sem):
        wid = lax.axis_index("s") * info.num_cores + lax.axis_index("c")
        base = wid * bpw
        for ci in range(nch):
            pltpu.sync_copy(idx_hbm.at[pl.ds(base + ci * ch, ch)],
                            idx_v.at[ci])
            pltpu.async_copy(table_hbm.at[idx_v.at[ci]], rows_v, sem).wait()
            pltpu.sync_copy(rows_v, out_hbm.at[pl.ds(base + ci * ch, ch)])

    return gather_kernel


def kernel(inputs, codebook):
    b, c, h, w = inputs.shape
    n = h * w
    batch = b * n
    x = jnp.transpose(inputs, (0, 2, 3, 1)).reshape(batch, c)
    x2 = jnp.sum(x * x, axis=-1, keepdims=True)
    e2 = jnp.sum(codebook * codebook, axis=-1).reshape(1, codebook.shape[0])
    xm2 = (-2.0 * x).astype(jnp.bfloat16)
    cbb = codebook.astype(jnp.bfloat16)
    idx2d, csum = _argmin_call(xm2, cbb, x2, e2)
    q = _make_gather(batch)(codebook, idx2d.reshape(batch))
    quantized = jnp.transpose(q.reshape(b, h, w, c), (0, 3, 1, 2))
    commit = csum[0, 0] / (batch * c)
    return quantized, commit


# speedup vs baseline: 1.7434x; 1.0336x over previous
import functools

import jax
import jax.numpy as jnp
from jax import lax
from jax.experimental import pallas as pl
from jax.experimental.pallas import tpu as pltpu
from jax.experimental.pallas import tpu_sc as plsc

K = 8192
D = 256
BT = 1024


_CHUNK = 2816


def _argmin_body(xm2_ref, cb_ref, x2_ref, e2_ref, idx_ref, commit_ref):
    t = pl.program_id(0)
    dots2 = lax.dot_general(xm2_ref[...], cb_ref[...],
                            (((1,), (1,)), ((), ())),
                            preferred_element_type=jnp.float32)
    kk = dots2.shape[1]
    x2v = x2_ref[...]
    e2v = e2_ref[...]
    nl = 128
    bounds = [(lo, min(lo + _CHUNK, kk)) for lo in range(0, kk, _CHUNK)]
    rv_cmp = None
    ri = None
    rexact = None
    for lo, hi in bounds:
        rv = rj = None
        for j in range((hi - lo) // nl):
            s = lo + j * nl
            ddj = x2v + dots2[:, s:s + nl] + e2v[:, s:s + nl]
            if rv is None:
                rv, rj = ddj, jnp.zeros_like(ddj)
            else:
                m = ddj < rv
                rv = jnp.where(m, ddj, rv)
                rj = jnp.where(m, jnp.float32(j), rj)
        lmin = jnp.min(rv, axis=1, keepdims=True)
        lane = lax.broadcasted_iota(jnp.int32, rv.shape, 1).astype(jnp.float32)
        gidx = rj * float(nl) + lane + float(lo)
        lidx = jnp.min(jnp.where(rv == lmin, gidx, float(kk)),
                       axis=1, keepdims=True)
        if rv_cmp is None:
            rv_cmp, ri, rexact = lmin.astype(jnp.bfloat16), lidx, lmin
        else:
            rvf = rv_cmp.astype(jnp.float32)
            take = lmin < rvf
            ri = jnp.where(take, lidx, ri)
            rexact = jnp.where(take, lmin, rexact)
            rv_cmp = jnp.where(take, lmin, rvf).astype(jnp.bfloat16)
    idx_ref[...] = ri.astype(jnp.int32)
    s = jnp.sum(rexact)

    @pl.when(t == 0)
    def _():
        commit_ref[0, 0] = s

    @pl.when(t > 0)
    def _():
        commit_ref[0, 0] = commit_ref[0, 0] + s


def _argmin_call(xm2, cb, x2, e2, bt=BT, interpret=False):
    ntok, d = xm2.shape
    kk = cb.shape[0]
    nt = ntok // bt
    return pl.pallas_call(
        _argmin_body,
        grid=(nt,),
        in_specs=[
            pl.BlockSpec((bt, d), lambda t: (t, 0)),
            pl.BlockSpec((kk, d), lambda t: (0, 0)),
            pl.BlockSpec((bt, 1), lambda t: (t, 0)),
            pl.BlockSpec((1, kk), lambda t: (0, 0)),
        ],
        out_specs=[
            pl.BlockSpec((bt, 1), lambda t: (t, 0)),
            pl.BlockSpec((1, 1), lambda t: (0, 0),
                         memory_space=pltpu.SMEM),
        ],
        out_shape=[
            jax.ShapeDtypeStruct((ntok, 1), jnp.int32),
            jax.ShapeDtypeStruct((1, 1), jnp.float32),
        ],
        interpret=interpret,
    )(xm2, cb, x2, e2)


@functools.cache
def _make_gather(batch):
    info = plsc.get_sparse_core_info()
    nw = info.num_cores * info.num_subcores
    bpw = batch // nw
    ch = min(bpw, 128)
    nch = bpw // ch
    mesh = plsc.VectorSubcoreMesh(core_axis_name="c", subcore_axis_name="s")

    @functools.partial(
        pl.kernel, mesh=mesh,
        out_type=jax.ShapeDtypeStruct((batch, D), jnp.float32),
        scratch_types=[
            pltpu.VMEM((nch, ch), jnp.int32),
            pltpu.VMEM((ch, D), jnp.float32),
            pltpu.VMEM((ch, D), jnp.float32),
            pltpu.SemaphoreType.DMA,
            pltpu.SemaphoreType.DMA,
        ],
    )
    def gather_kernel(table_hbm, idx_hbm, out_hbm, idx_v, rows0, rows1,
                      gsem, ssem):
        wid = lax.axis_index("s") * info.num_cores + lax.axis_index("c")
        base = wid * bpw
        bufs = [rows0, rows1]
        pltpu.sync_copy(idx_hbm.at[pl.ds(base, ch)], idx_v.at[0])
        g = pltpu.async_copy(table_hbm.at[idx_v.at[0]], bufs[0], gsem)
        scat = []
        for ci in range(nch):
            g.wait()
            scat.append(pltpu.async_copy(
                bufs[ci % 2], out_hbm.at[pl.ds(base + ci * ch, ch)], ssem))
            nxt = ci + 1
            if nxt < nch:
                pltpu.sync_copy(idx_hbm.at[pl.ds(base + nxt * ch, ch)],
                                idx_v.at[nxt])
                if nxt >= 2:
                    scat[nxt - 2].wait()
                g = pltpu.async_copy(table_hbm.at[idx_v.at[nxt]],
                                     bufs[nxt % 2], gsem)
        for s in scat[max(0, nch - 2):]:
            s.wait()

    return gather_kernel


def kernel(inputs, codebook):
    b, c, h, w = inputs.shape
    n = h * w
    batch = b * n
    x = jnp.transpose(inputs, (0, 2, 3, 1)).reshape(batch, c)
    x2 = jnp.sum(x * x, axis=-1, keepdims=True)
    e2 = jnp.sum(codebook * codebook, axis=-1).reshape(1, codebook.shape[0])
    xm2 = (-2.0 * x).astype(jnp.bfloat16)
    cbb = codebook.astype(jnp.bfloat16)
    idx2d, csum = _argmin_call(xm2, cbb, x2, e2)
    q = _make_gather(batch)(codebook, idx2d.reshape(batch))
    quantized = jnp.transpose(q.reshape(b, h, w, c), (0, 3, 1, 2))
    commit = csum[0, 0] / (batch * c)
    return quantized, commit


# speedup vs baseline: 1.9309x; 1.1075x over previous
import functools

import jax
import jax.numpy as jnp
from jax import lax
from jax.experimental import pallas as pl
from jax.experimental.pallas import tpu as pltpu
from jax.experimental.pallas import tpu_sc as plsc

K = 8192
D = 256
BT = 1024


_CHUNK = 2816


def _argmin_body(x_ref, cb_ref, e2_ref, idx_ref, commit_ref, cbb_ref):
    t = pl.program_id(0)

    @pl.when(t == 0)
    def _():
        cbb_ref[...] = cb_ref[...].astype(jnp.bfloat16)

    xv = x_ref[...]
    x2v = jnp.sum(xv * xv, axis=1, keepdims=True)
    xm2 = (-2.0 * xv).astype(jnp.bfloat16)
    dots2 = lax.dot_general(xm2, cbb_ref[...],
                            (((1,), (1,)), ((), ())),
                            preferred_element_type=jnp.float32)
    kk = dots2.shape[1]
    e2v = e2_ref[...]
    nl = 128
    bounds = [(lo, min(lo + _CHUNK, kk)) for lo in range(0, kk, _CHUNK)]
    rv_cmp = None
    ri = None
    rexact = None
    for lo, hi in bounds:
        rv = rj = None
        for j in range((hi - lo) // nl):
            s = lo + j * nl
            ddj = x2v + dots2[:, s:s + nl] + e2v[:, s:s + nl]
            if rv is None:
                rv, rj = ddj, jnp.zeros_like(ddj)
            else:
                m = ddj < rv
                rv = jnp.where(m, ddj, rv)
                rj = jnp.where(m, jnp.float32(j), rj)
        lmin = jnp.min(rv, axis=1, keepdims=True)
        lane = lax.broadcasted_iota(jnp.int32, rv.shape, 1).astype(jnp.float32)
        gidx = rj * float(nl) + lane + float(lo)
        lidx = jnp.min(jnp.where(rv == lmin, gidx, float(kk)),
                       axis=1, keepdims=True)
        if rv_cmp is None:
            rv_cmp, ri, rexact = lmin.astype(jnp.bfloat16), lidx, lmin
        else:
            rvf = rv_cmp.astype(jnp.float32)
            take = lmin < rvf
            ri = jnp.where(take, lidx, ri)
            rexact = jnp.where(take, lmin, rexact)
            rv_cmp = jnp.where(take, lmin, rvf).astype(jnp.bfloat16)
    idx_ref[...] = ri.astype(jnp.int32)
    s = jnp.sum(rexact)

    @pl.when(t == 0)
    def _():
        commit_ref[0, 0] = s

    @pl.when(t > 0)
    def _():
        commit_ref[0, 0] = commit_ref[0, 0] + s


def _argmin_call(x, cb, e2, bt=BT, interpret=False):
    ntok, d = x.shape
    kk = cb.shape[0]
    nt = ntok // bt
    return pl.pallas_call(
        _argmin_body,
        grid=(nt,),
        in_specs=[
            pl.BlockSpec((bt, d), lambda t: (t, 0)),
            pl.BlockSpec((kk, d), lambda t: (0, 0)),
            pl.BlockSpec((1, kk), lambda t: (0, 0)),
        ],
        out_specs=[
            pl.BlockSpec((bt, 1), lambda t: (t, 0)),
            pl.BlockSpec((1, 1), lambda t: (0, 0),
                         memory_space=pltpu.SMEM),
        ],
        out_shape=[
            jax.ShapeDtypeStruct((ntok, 1), jnp.int32),
            jax.ShapeDtypeStruct((1, 1), jnp.float32),
        ],
        scratch_shapes=[
            pltpu.VMEM((kk, d), jnp.bfloat16),
        ],
        interpret=interpret,
    )(x, cb, e2)


@functools.cache
def _make_gather(batch):
    info = plsc.get_sparse_core_info()
    nw = info.num_cores * info.num_subcores
    bpw = batch // nw
    ch = min(bpw, 128)
    nch = bpw // ch
    mesh = plsc.VectorSubcoreMesh(core_axis_name="c", subcore_axis_name="s")

    @functools.partial(
        pl.kernel, mesh=mesh,
        out_type=jax.ShapeDtypeStruct((batch, D), jnp.float32),
        scratch_types=[
            pltpu.VMEM((nch, ch), jnp.int32),
            pltpu.VMEM((ch, D), jnp.float32),
            pltpu.VMEM((ch, D), jnp.float32),
            pltpu.SemaphoreType.DMA,
            pltpu.SemaphoreType.DMA,
        ],
    )
    def gather_kernel(table_hbm, idx_hbm, out_hbm, idx_v, rows0, rows1,
                      gsem, ssem):
        wid = lax.axis_index("s") * info.num_cores + lax.axis_index("c")
        base = wid * bpw
        bufs = [rows0, rows1]
        pltpu.sync_copy(idx_hbm.at[pl.ds(base, ch)], idx_v.at[0])
        g = pltpu.async_copy(table_hbm.at[idx_v.at[0]], bufs[0], gsem)
        scat = []
        for ci in range(nch):
            g.wait()
            scat.append(pltpu.async_copy(
                bufs[ci % 2], out_hbm.at[pl.ds(base + ci * ch, ch)], ssem))
            nxt = ci + 1
            if nxt < nch:
                pltpu.sync_copy(idx_hbm.at[pl.ds(base + nxt * ch, ch)],
                                idx_v.at[nxt])
                if nxt >= 2:
                    scat[nxt - 2].wait()
                g = pltpu.async_copy(table_hbm.at[idx_v.at[nxt]],
                                     bufs[nxt % 2], gsem)
        for s in scat[max(0, nch - 2):]:
            s.wait()

    return gather_kernel


def kernel(inputs, codebook):
    b, c, h, w = inputs.shape
    n = h * w
    batch = b * n
    x = jnp.transpose(inputs, (0, 2, 3, 1)).reshape(batch, c)
    e2 = jnp.sum(codebook * codebook, axis=-1).reshape(1, codebook.shape[0])
    idx2d, csum = _argmin_call(x, codebook, e2)
    q = _make_gather(batch)(codebook, idx2d.reshape(batch))
    quantized = jnp.transpose(q.reshape(b, h, w, c), (0, 3, 1, 2))
    commit = csum[0, 0] / (batch * c)
    return quantized, commit
